# baseline (speedup 1.0000x reference)
import functools

import jax
import jax.numpy as jnp
from jax import lax
from jax.experimental import pallas as pl
from jax.experimental.pallas import tpu as pltpu
from jax.experimental.pallas import tpu_sc as plsc

_I_T = 512

_T = 256
_K = 2
_E = 8


def _routing_sc_kernel(idx_hbm, w_hbm, out_hbm, idx_v, w_v, acc_v):
    nc = 2
    wid = lax.axis_index("s") * nc + lax.axis_index("c")
    base = wid * 16

    pltpu.sync_copy(idx_hbm.at[pl.ds(base, 16)], idx_v)
    pltpu.sync_copy(w_hbm.at[pl.ds(base, 16)], w_v)

    for j in range(4):
        acc_v[pl.ds(j * 16, 16)] = jnp.zeros((16,), jnp.float32)

    lane = lax.iota(jnp.int32, 16)
    tok_local = lax.shift_right_logical(lane, 1)
    tgt = tok_local * _E + idx_v[...]
    even = (lane & 1) == 0
    plsc.addupdate_scatter(acc_v, [tgt], w_v[...], mask=even)
    plsc.addupdate_scatter(acc_v, [tgt], w_v[...], mask=jnp.logical_not(even))

    pltpu.sync_copy(acc_v, out_hbm.at[pl.ds(wid * 64, 64)])


def _routing_weights_sc(idx_flat, w_flat):
    mesh = plsc.VectorSubcoreMesh(core_axis_name="c", subcore_axis_name="s")
    kfn = functools.partial(
        pl.kernel,
        mesh=mesh,
        out_type=jax.ShapeDtypeStruct((_T * _E,), jnp.float32),
        scratch_types=[
            pltpu.VMEM((16,), jnp.int32),
            pltpu.VMEM((16,), jnp.float32),
            pltpu.VMEM((64,), jnp.float32),
        ],
        compiler_params=pltpu.CompilerParams(needs_layout_passes=False),
    )(_routing_sc_kernel)
    return kfn(idx_flat, w_flat).reshape(_T, _E)


def _moe_kernel(wd_ref, x_ref, g_ref, u_ref, d_ref, out_ref):
    e = pl.program_id(0)
    i = pl.program_id(1)

    @pl.when((e == 0) & (i == 0))
    def _init():
        out_ref[...] = jnp.zeros_like(out_ref)

    x = x_ref[...]
    g = g_ref[0].astype(jnp.bfloat16)
    u = u_ref[0].astype(jnp.bfloat16)
    d = d_ref[0].astype(jnp.bfloat16)

    dn = (((1,), (1,)), ((), ()))
    gate = jax.lax.dot_general(x, g, dn, preferred_element_type=jnp.float32)
    up = jax.lax.dot_general(x, u, dn, preferred_element_type=jnp.float32)
    inter = (gate * jax.lax.logistic(gate) * up).astype(jnp.bfloat16)
    part = jax.lax.dot_general(inter, d, dn, preferred_element_type=jnp.float32)

    col = jax.lax.broadcasted_iota(jnp.int32, wd_ref.shape, 1)
    w_e = jnp.sum(jnp.where(col == e, wd_ref[...], 0.0), axis=1,
                  keepdims=True)
    out_ref[...] += w_e * part


def kernel(x, expert_indices, expert_weights, gate_proj, up_proj, down_proj):
    b, s, h = x.shape
    t = b * s
    e, i_dim, _ = gate_proj.shape

    x_flat = x.reshape(t, h).astype(jnp.bfloat16)
    idx_flat = expert_indices.reshape(t * _K)
    w_flat = expert_weights.reshape(t * _K).astype(jnp.float32)

    w_dense = _routing_weights_sc(idx_flat, w_flat)

    grid = (e, i_dim // _I_T)
    out = pl.pallas_call(
        _moe_kernel,
        grid=grid,
        in_specs=[
            pl.BlockSpec((t, _E), lambda ei, ii: (0, 0)),
            pl.BlockSpec((t, h), lambda ei, ii: (0, 0)),
            pl.BlockSpec((1, _I_T, h), lambda ei, ii: (ei, ii, 0)),
            pl.BlockSpec((1, _I_T, h), lambda ei, ii: (ei, ii, 0)),
            pl.BlockSpec((1, h, _I_T), lambda ei, ii: (ei, 0, ii)),
        ],
        out_specs=pl.BlockSpec((t, h), lambda ei, ii: (0, 0)),
        out_shape=jax.ShapeDtypeStruct((t, h), jnp.float32),
    )(w_dense, x_flat, gate_proj, up_proj, down_proj)
    return out.reshape(b, s, h)


# speedup vs baseline: 1.0565x; 1.0565x over previous
import jax
import jax.numpy as jnp
from jax.experimental import pallas as pl

_I_T = 512


def _moe_kernel(idx_ref, w_ref, x_ref, g_ref, u_ref, d_ref, out_ref):
    e = pl.program_id(0)
    i = pl.program_id(1)

    @pl.when((e == 0) & (i == 0))
    def _init():
        out_ref[...] = jnp.zeros_like(out_ref)

    x = x_ref[...]
    g = g_ref[0].astype(jnp.bfloat16)
    u = u_ref[0].astype(jnp.bfloat16)
    d = d_ref[0].astype(jnp.bfloat16)

    dn = (((1,), (1,)), ((), ()))
    gate = jax.lax.dot_general(x, g, dn, preferred_element_type=jnp.float32)
    up = jax.lax.dot_general(x, u, dn, preferred_element_type=jnp.float32)
    inter = (gate * jax.lax.logistic(gate) * up).astype(jnp.bfloat16)
    part = jax.lax.dot_general(inter, d, dn, preferred_element_type=jnp.float32)

    w_e = jnp.sum(jnp.where(idx_ref[...] == e, w_ref[...], 0.0), axis=1,
                  keepdims=True)
    out_ref[...] += w_e * part


def kernel(x, expert_indices, expert_weights, gate_proj, up_proj, down_proj):
    b, s, h = x.shape
    t = b * s
    e, i_dim, _ = gate_proj.shape
    k = expert_indices.shape[-1]

    x_flat = x.reshape(t, h).astype(jnp.bfloat16)
    idx = expert_indices.reshape(t, k)
    w = expert_weights.reshape(t, k).astype(jnp.float32)

    grid = (e, i_dim // _I_T)
    out = pl.pallas_call(
        _moe_kernel,
        grid=grid,
        in_specs=[
            pl.BlockSpec((t, k), lambda ei, ii: (0, 0)),
            pl.BlockSpec((t, k), lambda ei, ii: (0, 0)),
            pl.BlockSpec((t, h), lambda ei, ii: (0, 0)),
            pl.BlockSpec((1, _I_T, h), lambda ei, ii: (ei, ii, 0)),
            pl.BlockSpec((1, _I_T, h), lambda ei, ii: (ei, ii, 0)),
            pl.BlockSpec((1, h, _I_T), lambda ei, ii: (ei, 0, ii)),
        ],
        out_specs=pl.BlockSpec((t, h), lambda ei, ii: (0, 0)),
        out_shape=jax.ShapeDtypeStruct((t, h), jnp.float32),
    )(idx, w, x_flat, gate_proj, up_proj, down_proj)
    return out.reshape(b, s, h)
